# baseline (speedup 1.0000x reference)
import functools

import jax
import jax.numpy as jnp
from jax import lax
from jax.experimental import pallas as pl
from jax.experimental.pallas import tpu as pltpu
from jax.experimental.pallas import tpu_sc as plsc

_NUM_TILES = 32
_GROUP = 128
_B = 16384 * 50
_D = 32
_GROUPS_PER_TILE = _B // (_NUM_TILES * _GROUP)
_NSLOT = 4
_NSTEP = _GROUPS_PER_TILE // _NSLOT

_mesh = plsc.VectorSubcoreMesh(core_axis_name="c", subcore_axis_name="s")


@functools.partial(
    pl.kernel,
    mesh=_mesh,
    compiler_params=pltpu.CompilerParams(use_tc_tiling_on_sc=False),
    out_type=jax.ShapeDtypeStruct((_B, _D), jnp.float32),
    scratch_types=(
        [pltpu.VMEM((_GROUPS_PER_TILE, _GROUP), jnp.int32),
         pltpu.VMEM((2 * _NSLOT, _GROUP, _D), jnp.float32)]
        + [pltpu.SemaphoreType.DMA] * (4 * _NSLOT)
    ),
)
def _gather_kernel(idx_hbm, table_hbm, out_hbm, idx_v, rows_v, *sems):
    gsems = sems[:2 * _NSLOT]
    osems = sems[2 * _NSLOT:]
    wid = lax.axis_index("s") * 2 + lax.axis_index("c")
    gbase = wid * _GROUPS_PER_TILE
    pltpu.sync_copy(idx_hbm.at[pl.ds(gbase, _GROUPS_PER_TILE)], idx_v)

    def _gather(g, b):
        return pltpu.make_async_copy(
            table_hbm.at[idx_v.at[g]], rows_v.at[b], gsems[b])

    def _out(g, b):
        return pltpu.make_async_copy(
            rows_v.at[b], out_hbm.at[pl.ds((gbase + g) * _GROUP, _GROUP)],
            osems[b])

    def _start_gathers(s, h):
        for j in range(_NSLOT):
            _gather(s * _NSLOT + j, h * _NSLOT + j).start()

    def _drain_gathers_start_outs(s, h):
        for j in range(_NSLOT):
            b = h * _NSLOT + j
            _gather(s * _NSLOT + j, b).wait()
            _out(s * _NSLOT + j, b).start()

    def _wait_outs(s, h):
        for j in range(_NSLOT):
            _out(s * _NSLOT + j, h * _NSLOT + j).wait()

    _start_gathers(0, 0)
    _start_gathers(1, 1)
    _drain_gathers_start_outs(0, 0)

    @pl.loop(0, (_NSTEP - 2) // 2)
    def _steps(it):
        sa = 2 + 2 * it
        _wait_outs(sa - 2, 0)
        _start_gathers(sa, 0)
        _drain_gathers_start_outs(sa - 1, 1)
        sb = sa + 1
        _wait_outs(sb - 2, 1)
        _start_gathers(sb, 1)
        _drain_gathers_start_outs(sb - 1, 0)

    _drain_gathers_start_outs(_NSTEP - 1, 1)
    _wait_outs(_NSTEP - 2, 0)
    _wait_outs(_NSTEP - 1, 1)


_V = 1000000
_CBLK = 2048
_RBLK = _CBLK * _D // 128


def _tc_pack_body(x_ref, y_ref):
    z = x_ref[...].T
    y_ref[...] = z.reshape(_RBLK, 4, _D)


def _tc_pack(wt):
    return pl.pallas_call(
        _tc_pack_body,
        grid=((_V + _CBLK - 1) // _CBLK,),
        in_specs=[pl.BlockSpec((_D, _CBLK), lambda i: (0, i))],
        out_specs=pl.BlockSpec((_RBLK, 4, _D), lambda i: (i, 0, 0)),
        out_shape=jax.ShapeDtypeStruct((_V * _D // 128, 4, _D), jnp.float32),
    )(wt)


def kernel(input, weight):
    idx = input.reshape(-1).astype(jnp.int32).reshape(_B // _GROUP, _GROUP)
    table = _tc_pack(weight.T).reshape(_V, _D)
    out = _gather_kernel(idx, table)
    return out.reshape(tuple(input.shape) + tuple(weight.shape[1:]))


# speedup vs baseline: 1.2028x; 1.2028x over previous
import functools

import jax
import jax.numpy as jnp
from jax import lax
from jax.experimental import pallas as pl
from jax.experimental.pallas import tpu as pltpu
from jax.experimental.pallas import tpu_sc as plsc

_NUM_TILES = 32
_GROUP = 128
_B = 16384 * 50
_D = 32
_GROUPS_PER_TILE = _B // (_NUM_TILES * _GROUP)
_NSLOT = 4
_NSTEP = _GROUPS_PER_TILE // _NSLOT

_mesh = plsc.VectorSubcoreMesh(core_axis_name="c", subcore_axis_name="s")


@functools.partial(
    pl.kernel,
    mesh=_mesh,
    compiler_params=pltpu.CompilerParams(use_tc_tiling_on_sc=False),
    out_type=jax.ShapeDtypeStruct((_B, _D), jnp.float32),
    scratch_types=(
        [pltpu.VMEM((_GROUPS_PER_TILE, _GROUP), jnp.int32),
         pltpu.VMEM((2 * _NSLOT, _GROUP, _D), jnp.float32)]
        + [pltpu.SemaphoreType.DMA] * (4 * _NSLOT)
    ),
)
def _gather_kernel(idx_hbm, table_hbm, out_hbm, idx_v, rows_v, *sems):
    gsems = sems[:2 * _NSLOT]
    osems = sems[2 * _NSLOT:]
    wid = lax.axis_index("s") * 2 + lax.axis_index("c")
    gbase = wid * _GROUPS_PER_TILE
    pltpu.sync_copy(idx_hbm.at[pl.ds(gbase, _GROUPS_PER_TILE)], idx_v)

    def _gather(g, b):
        return pltpu.make_async_copy(
            table_hbm.at[idx_v.at[g]], rows_v.at[b], gsems[b])

    def _out(g, b):
        return pltpu.make_async_copy(
            rows_v.at[b], out_hbm.at[pl.ds((gbase + g) * _GROUP, _GROUP)],
            osems[b])

    def _start_gathers(s, h):
        for j in range(_NSLOT):
            _gather(s * _NSLOT + j, h * _NSLOT + j).start()

    def _drain_gathers_start_outs(s, h):
        for j in range(_NSLOT):
            b = h * _NSLOT + j
            _gather(s * _NSLOT + j, b).wait()
            _out(s * _NSLOT + j, b).start()

    def _wait_outs(s, h):
        for j in range(_NSLOT):
            _out(s * _NSLOT + j, h * _NSLOT + j).wait()

    _start_gathers(0, 0)
    _start_gathers(1, 1)
    _drain_gathers_start_outs(0, 0)

    @pl.loop(0, (_NSTEP - 2) // 2)
    def _steps(it):
        sa = 2 + 2 * it
        _wait_outs(sa - 2, 0)
        _start_gathers(sa, 0)
        _drain_gathers_start_outs(sa - 1, 1)
        sb = sa + 1
        _wait_outs(sb - 2, 1)
        _start_gathers(sb, 1)
        _drain_gathers_start_outs(sb - 1, 0)

    _drain_gathers_start_outs(_NSTEP - 1, 1)
    _wait_outs(_NSTEP - 2, 0)
    _wait_outs(_NSTEP - 1, 1)


_V = 1000000
_CBLK = 2048
_RBLK = _CBLK * _D // 128


def _tc_pack_body(x_ref, y_ref):
    y_ref[:, 0:_D] = x_ref[...].T


def _tc_pack(wt):
    return pl.pallas_call(
        _tc_pack_body,
        grid=((_V + _CBLK - 1) // _CBLK,),
        in_specs=[pl.BlockSpec((_D, _CBLK), lambda i: (0, i))],
        out_specs=pl.BlockSpec((_CBLK, 128), lambda i: (i, 0)),
        out_shape=jax.ShapeDtypeStruct((_V, 128), jnp.float32),
    )(wt)


def kernel(input, weight):
    idx = (input.reshape(-1).astype(jnp.int32) * 4).reshape(_B // _GROUP, _GROUP)
    table = _tc_pack(weight.T).reshape(_V * 4, _D)
    out = _gather_kernel(idx, table)
    return out.reshape(tuple(input.shape) + tuple(weight.shape[1:]))
